# baseline (speedup 1.0000x reference)
import functools

import jax
import jax.numpy as jnp
from jax import lax
from jax.experimental import pallas as pl
from jax.experimental.pallas import tpu as pltpu
from jax.experimental.pallas import tpu_sc as plsc

B = 4096
N = 8192
NC = 2
NS = 16
L = 16
NW = NC * NS
ROWS_PER_W = B // NW
R = 4
CHUNKS = ROWS_PER_W // R


def _body(x_hbm, idx_hbm, out_hbm, idx_v, in_v, out_v):
    wid = lax.axis_index("s") * NC + lax.axis_index("c")
    row0 = wid * ROWS_PER_W

    pltpu.sync_copy(idx_hbm, idx_v)

    def chunk_body(c, carry):
        r0 = row0 + c * R
        for r in range(R):
            pltpu.sync_copy(x_hbm.at[r0 + r], in_v.at[pl.ds(r * N, N)])

        def col_body(j, carry2):
            jj = j * L
            idx16 = idx_v[pl.ds(jj, L)]
            for r in range(R):
                val = plsc.load_gather(in_v, [idx16 + (r * N)])
                out_v[pl.ds(r * N + jj, L)] = val
            return carry2

        lax.fori_loop(0, N // L, col_body, 0, unroll=4)
        for r in range(R):
            pltpu.sync_copy(out_v.at[pl.ds(r * N, N)], out_hbm.at[r0 + r])
        return carry

    lax.fori_loop(0, CHUNKS, chunk_body, 0)


@jax.jit
def kernel(x, ind_rate_matching):
    mesh = plsc.VectorSubcoreMesh(core_axis_name="c", subcore_axis_name="s")
    return pl.kernel(
        _body,
        out_type=jax.ShapeDtypeStruct((B, N), jnp.float32),
        mesh=mesh,
        scratch_types=[
            pltpu.VMEM((N,), jnp.int32),
            pltpu.VMEM((R * N,), jnp.float32),
            pltpu.VMEM((R * N,), jnp.float32),
        ],
        compiler_params=pltpu.CompilerParams(needs_layout_passes=False),
    )(x, ind_rate_matching)


# speedup vs baseline: 1.3542x; 1.3542x over previous
import jax
import jax.numpy as jnp
from jax import lax
from jax.experimental import pallas as pl
from jax.experimental.pallas import tpu as pltpu
from jax.experimental.pallas import tpu_sc as plsc

B = 4096
N = 8192
NC = 2
NS = 16
L = 16
NW = NC * NS
ROWS_PER_W = B // NW
R = 2
CHUNKS = ROWS_PER_W // R
CW = R * N


def _body(x_hbm, idx_hbm, out_hbm, idx_v, in0, in1, out0, out1,
          sin0, sin1, sout0, sout1):
    wid = lax.axis_index("s") * NC + lax.axis_index("c")
    base = wid * (ROWS_PER_W * N)

    pltpu.sync_copy(idx_hbm, idx_v)

    def in_copy(c, buf, sem):
        return pltpu.make_async_copy(
            x_hbm.at[pl.ds(base + c * CW, CW)], buf, sem)

    def out_copy(c, buf, sem):
        return pltpu.make_async_copy(
            buf, out_hbm.at[pl.ds(base + c * CW, CW)], sem)

    def compute(inb, outb):
        @plsc.parallel_loop(0, N // L, 1, unroll=8)
        def _(j):
            jj = j * L
            idx16 = idx_v[pl.ds(jj, L)]
            for r in range(R):
                outb[pl.ds(r * N + jj, L)] = plsc.load_gather(
                    inb, [idx16 + (r * N)])

    in_copy(0, in0, sin0).start()
    in_copy(1, in1, sin1).start()

    def pair_body(p, carry):
        g0 = p * 2
        for g, inb, outb, sin, sout in (
                (g0, in0, out0, sin0, sout0),
                (g0 + 1, in1, out1, sin1, sout1)):
            in_copy(g, inb, sin).wait()

            @pl.when(p > 0)
            def _():
                out_copy(g - 2, outb, sout).wait()

            compute(inb, outb)
            out_copy(g, outb, sout).start()

            @pl.when(g + 2 < CHUNKS)
            def _():
                in_copy(g + 2, inb, sin).start()
        return carry

    lax.fori_loop(0, CHUNKS // 2, pair_body, 0)
    out_copy(CHUNKS - 2, out0, sout0).wait()
    out_copy(CHUNKS - 1, out1, sout1).wait()


@jax.jit
def kernel(x, ind_rate_matching):
    mesh = plsc.VectorSubcoreMesh(core_axis_name="c", subcore_axis_name="s")
    out_flat = pl.kernel(
        _body,
        out_type=jax.ShapeDtypeStruct((B * N,), jnp.float32),
        mesh=mesh,
        scratch_types=[
            pltpu.VMEM((N,), jnp.int32),
            pltpu.VMEM((CW,), jnp.float32),
            pltpu.VMEM((CW,), jnp.float32),
            pltpu.VMEM((CW,), jnp.float32),
            pltpu.VMEM((CW,), jnp.float32),
            pltpu.SemaphoreType.DMA,
            pltpu.SemaphoreType.DMA,
            pltpu.SemaphoreType.DMA,
            pltpu.SemaphoreType.DMA,
        ],
        compiler_params=pltpu.CompilerParams(needs_layout_passes=False),
    )(x.reshape(B * N), ind_rate_matching)
    return out_flat.reshape(B, N)


# speedup vs baseline: 3.9812x; 2.9400x over previous
import jax
import jax.numpy as jnp
from jax import lax
from jax.experimental import pallas as pl
from jax.experimental.pallas import tpu as pltpu
from jax.experimental.pallas import tpu_sc as plsc

B = 4096
N = 8192
NC = 2
NS = 16
L = 16
NW = NC * NS
ROWS_PER_W = B // NW
R = 2
CHUNKS = ROWS_PER_W // R


def _body(x_hbm, idx_hbm, out_hbm, idx_v, in0, in1, out0, out1,
          sin0, sin1, sout0, sout1):
    wid = lax.axis_index("s") * NC + lax.axis_index("c")
    row_base = wid * ROWS_PER_W

    pltpu.sync_copy(idx_hbm, idx_v)

    def in_copy(c, buf, sem):
        return pltpu.make_async_copy(
            x_hbm.at[pl.ds(row_base + c * R, R)], buf, sem)

    def out_copy(c, buf, sem):
        return pltpu.make_async_copy(
            buf, out_hbm.at[pl.ds(row_base + c * R, R)], sem)

    def compute(inb, outb):
        @plsc.parallel_loop(0, N // L, 1, unroll=8)
        def _(j):
            jj = j * L
            idx16 = idx_v[pl.ds(jj, L)]
            for r in range(R):
                row16 = jnp.full((L,), r, jnp.int32)
                outb[r, pl.ds(jj, L)] = plsc.load_gather(inb, [row16, idx16])

    in_copy(0, in0, sin0).start()
    in_copy(1, in1, sin1).start()

    def pair_body(p, carry):
        g0 = p * 2
        for g, inb, outb, sin, sout in (
                (g0, in0, out0, sin0, sout0),
                (g0 + 1, in1, out1, sin1, sout1)):
            in_copy(g, inb, sin).wait()

            @pl.when(p > 0)
            def _():
                out_copy(g - 2, outb, sout).wait()

            compute(inb, outb)
            out_copy(g, outb, sout).start()

            @pl.when(g + 2 < CHUNKS)
            def _():
                in_copy(g + 2, inb, sin).start()
        return carry

    lax.fori_loop(0, CHUNKS // 2, pair_body, 0)
    out_copy(CHUNKS - 2, out0, sout0).wait()
    out_copy(CHUNKS - 1, out1, sout1).wait()


@jax.jit
def kernel(x, ind_rate_matching):
    mesh = plsc.VectorSubcoreMesh(core_axis_name="c", subcore_axis_name="s")
    return pl.kernel(
        _body,
        out_type=jax.ShapeDtypeStruct((B, N), jnp.float32),
        mesh=mesh,
        scratch_types=[
            pltpu.VMEM((N,), jnp.int32),
            pltpu.VMEM((R, N), jnp.float32),
            pltpu.VMEM((R, N), jnp.float32),
            pltpu.VMEM((R, N), jnp.float32),
            pltpu.VMEM((R, N), jnp.float32),
            pltpu.SemaphoreType.DMA,
            pltpu.SemaphoreType.DMA,
            pltpu.SemaphoreType.DMA,
            pltpu.SemaphoreType.DMA,
        ],
        compiler_params=pltpu.CompilerParams(needs_layout_passes=False),
    )(x, ind_rate_matching)


# speedup vs baseline: 4.1311x; 1.0377x over previous
import jax
import jax.numpy as jnp
from jax import lax
from jax.experimental import pallas as pl
from jax.experimental.pallas import tpu as pltpu
from jax.experimental.pallas import tpu_sc as plsc

B = 4096
N = 8192
NC = 2
NS = 16
L = 16
NW = NC * NS
ROWS_PER_W = B // NW
R = 2
CHUNKS = ROWS_PER_W // R


NIN = 4
NOUT = 2


def _body(x_hbm, idx_hbm, out_hbm, idx_v, ins, outs, sins, souts):
    wid = lax.axis_index("s") * NC + lax.axis_index("c")
    row_base = wid * ROWS_PER_W

    pltpu.sync_copy(idx_hbm, idx_v)

    def in_copy(c, k):
        return pltpu.make_async_copy(
            x_hbm.at[pl.ds(row_base + c * R, R)], ins[k], sins[k])

    def out_copy(c, k):
        return pltpu.make_async_copy(
            outs[k], out_hbm.at[pl.ds(row_base + c * R, R)], souts[k])

    def compute(inb, outb):
        @plsc.parallel_loop(0, N // L, 1, unroll=8)
        def _(j):
            jj = j * L
            idx16 = idx_v[pl.ds(jj, L)]
            for r in range(R):
                row16 = jnp.full((L,), r, jnp.int32)
                outb[r, pl.ds(jj, L)] = plsc.load_gather(inb, [row16, idx16])

    for k in range(NIN):
        in_copy(k, k).start()

    def group_body(p, carry):
        g0 = p * NIN
        for k in range(NIN):
            g = g0 + k
            ko = k % NOUT
            in_copy(g, k).wait()

            @pl.when(g >= NOUT)
            def _():
                out_copy(g - NOUT, ko).wait()

            compute(ins[k], outs[ko])
            out_copy(g, ko).start()

            @pl.when(g + NIN < CHUNKS)
            def _():
                in_copy(g + NIN, k).start()
        return carry

    lax.fori_loop(0, CHUNKS // NIN, group_body, 0)
    for k in range(NOUT):
        out_copy(CHUNKS - NOUT + k, (CHUNKS - NOUT + k) % NOUT).wait()


@jax.jit
def kernel(x, ind_rate_matching):
    mesh = plsc.VectorSubcoreMesh(core_axis_name="c", subcore_axis_name="s")
    return pl.kernel(
        _body,
        out_type=jax.ShapeDtypeStruct((B, N), jnp.float32),
        mesh=mesh,
        scratch_types=[
            pltpu.VMEM((N,), jnp.int32),
            [pltpu.VMEM((R, N), jnp.float32) for _ in range(NIN)],
            [pltpu.VMEM((R, N), jnp.float32) for _ in range(NOUT)],
            [pltpu.SemaphoreType.DMA for _ in range(NIN)],
            [pltpu.SemaphoreType.DMA for _ in range(NOUT)],
        ],
        compiler_params=pltpu.CompilerParams(needs_layout_passes=False),
    )(x, ind_rate_matching)
